# Initial kernel scaffold; baseline (speedup 1.0000x reference)
import jax
import jax.numpy as jnp
from jax.experimental import pallas as pl


def kernel(epoch, pro_idx, hpo_idx, X_exp, X_esm, X_ppi, X_term, A_ppi_idx, A_ppi_val, A_rel_idx, A_rel_val, A_cop_idx, A_cop_val, params):
    raise NotImplementedError("write your pallas kernel here")


# speedup vs baseline: 1.2278x; 1.2278x over previous
import jax
import jax.numpy as jnp
from jax.experimental import pallas as pl

PRO = 10000
TERM = 5000
N = PRO + TERM
D = 256
B = 4096
TEMP = 0.1


def _bn(x, g, b):
    m = jnp.mean(x, axis=0)
    v = jnp.var(x, axis=0)
    return (x - m) / jnp.sqrt(v + 1e-5) * g + b


def _spmm(idx, val, x, n):
    return jax.ops.segment_sum(x[idx[1]] * val[:, None], idx[0], num_segments=n)


def _infonce(v1, v2, W, b, mask, n):
    v1 = v1 @ W + b
    v2 = v2 @ W + b
    v1 = v1 / jnp.linalg.norm(v1, axis=1, keepdims=True)
    v2 = v2 / jnp.linalg.norm(v2, axis=1, keepdims=True)
    pos = v1 @ v2.T / TEMP
    pos = jnp.where(mask[None, :], pos, -jnp.inf)
    score = jnp.diag(jax.nn.log_softmax(pos, axis=1))
    return -jnp.sum(jnp.where(mask, score, 0.0)) / n


def _copy_body(x_ref, o_ref):
    o_ref[...] = x_ref[...]


def _pallas_copy(x):
    return pl.pallas_call(
        _copy_body,
        out_shape=jax.ShapeDtypeStruct(x.shape, x.dtype),
        grid=(x.shape[0] // 1000,),
        in_specs=[pl.BlockSpec((1000, x.shape[1]), lambda i: (i, 0))],
        out_specs=pl.BlockSpec((1000, x.shape[1]), lambda i: (i, 0)),
    )(x)


def kernel(epoch, pro_idx, hpo_idx, X_exp, X_esm, X_ppi, X_term, A_ppi_idx, A_ppi_val, A_rel_idx, A_rel_val, A_cop_idx, A_cop_val, params):
    p = params
    pe = _bn(jax.nn.leaky_relu(X_exp @ p['W_exp'] + p['b_exp']), p['g_exp'], p['be_exp'])
    ps = _bn(jax.nn.leaky_relu(X_esm @ p['W_esm'] + p['b_esm']), p['g_esm'], p['be_esm'])
    pp = _bn(jax.nn.leaky_relu(X_ppi @ p['W_ppi'] + p['b_ppi']), p['g_ppi'], p['be_ppi'])
    t0 = _bn(jax.nn.leaky_relu(X_term @ p['W_pub0'] + p['b_pub0']), p['g_p0'], p['be_p0'])
    t1 = _bn(jax.nn.leaky_relu(X_term @ p['W_pub1'] + p['b_pub1']), p['g_p1'], p['be_p1'])
    t2 = _bn(jax.nn.leaky_relu(X_term @ p['W_pub2'] + p['b_pub2']), p['g_p2'], p['be_p2'])

    ego = jnp.concatenate([jnp.concatenate([pe, t0], axis=0),
                           jnp.concatenate([ps, t1], axis=0),
                           jnp.concatenate([pp, t2], axis=0)], axis=1)

    prop0 = _spmm(A_rel_idx, A_rel_val, ego, N)
    prop1 = _spmm(A_cop_idx, A_cop_val, ego, N)

    pe_f0, ps_f0, pp_f10 = prop0[:PRO, :D], prop0[:PRO, D:2 * D], prop0[:PRO, 2 * D:]
    te_f0, ts_f0, tp_f0 = prop0[PRO:, :D], prop0[PRO:, D:2 * D], prop0[PRO:, 2 * D:]
    pe_f1, ps_f1, pp_f11 = prop1[:PRO, :D], prop1[:PRO, D:2 * D], prop1[:PRO, 2 * D:]
    te_f1, ts_f1, tp_f1 = prop1[PRO:, :D], prop1[PRO:, D:2 * D], prop1[PRO:, 2 * D:]

    pp_f0 = _spmm(A_ppi_idx, A_ppi_val, pp_f10, PRO)
    pp_f1 = _spmm(A_ppi_idx, A_ppi_val, pp_f11, PRO)

    pset, pcnt = jnp.unique(pro_idx, size=B, fill_value=0, return_counts=True)
    hset, hcnt = jnp.unique(hpo_idx, size=B, fill_value=0, return_counts=True)
    pmask = pcnt > 0
    hmask = hcnt > 0
    pn = jnp.sum(pmask)
    hn = jnp.sum(hmask)
    lp = (_infonce(pe_f0[pset], pe_f1[pset], p['W_pp'], p['b_pp'], pmask, pn)
          + _infonce(ps_f0[pset], ps_f1[pset], p['W_pp'], p['b_pp'], pmask, pn)
          + _infonce(pp_f0[pset], pp_f1[pset], p['W_pp'], p['b_pp'], pmask, pn)) / 3.0
    lt = (_infonce(te_f0[hset], te_f1[hset], p['W_pt'], p['b_pt'], hmask, hn)
          + _infonce(ts_f0[hset], ts_f1[hset], p['W_pt'], p['b_pt'], hmask, hn)
          + _infonce(tp_f0[hset], tp_f1[hset], p['W_pt'], p['b_pt'], hmask, hn)) / 3.0

    pe_f0 = _pallas_copy(pe_f0)
    return (pe_f0, te_f0, ps_f0, ts_f0, pp_f0, tp_f0, pe, ps, pp, (lp + lt) / 2.0)


# speedup vs baseline: 1.8257x; 1.4870x over previous
import functools

import jax
import jax.numpy as jnp
from jax import lax
from jax.experimental import pallas as pl
from jax.experimental.pallas import tpu as pltpu
from jax.experimental.pallas import tpu_sc as plsc

PRO = 10000
TERM = 5000
N = PRO + TERM
D = 256
B = 4096
TEMP = 0.1

LANES = 16
NSC = 2
NTILES = 16
FCHUNK = 64
EDGE_BLK = 128

ROWS_PAD_N = 15104


def _spmm_body(tableH, srcH, dstH, valH, outH, acc, src_a, dst_a, val_a,
               rows0, rows1, gsem0, gsem1, *, rows_pad, nchunks, nb_tile):
    cpc = nchunks // NSC
    rpt = rows_pad // NTILES
    npt = nb_tile * EDGE_BLK
    cid = lax.axis_index("c")
    sid = lax.axis_index("s")

    pltpu.sync_copy(srcH.at[pl.ds(sid * npt, npt)], src_a)
    pltpu.sync_copy(dstH.at[sid], dst_a)
    pltpu.sync_copy(valH.at[pl.ds(sid * npt, npt)], val_a)

    def add_offset(delta):
        dvec = jnp.full((LANES,), delta, jnp.int32)

        def ob(i, _):
            sl = pl.ds(i * LANES, LANES)
            src_a[sl] = src_a[sl] + dvec
            return 0

        lax.fori_loop(0, npt // LANES, ob, 0)

    add_offset(cid * (cpc * rows_pad))

    def zero_rows0():
        def zb(r, _):
            for q in range(FCHUNK // LANES):
                rows0[r, pl.ds(q * LANES, LANES)] = jnp.zeros((LANES,), jnp.float32)
            return 0

        lax.fori_loop(0, EDGE_BLK, zb, 0)

    _gdn = lax.GatherDimensionNumbers(offset_dims=(), collapsed_slice_dims=(0,),
                                      start_index_map=(0,))

    def scale(rows, b):
        def gb(g, _):
            v16 = val_a[pl.ds(b * EDGE_BLK + g * LANES, LANES)]
            for k in range(LANES):
                vb = lax.gather(v16, jnp.full((LANES, 1), k, jnp.int32), _gdn,
                                slice_sizes=(1,),
                                mode=lax.GatherScatterMode.PROMISE_IN_BOUNDS)
                e = g * LANES + k
                for q in range(FCHUNK // LANES):
                    sl = pl.ds(q * LANES, LANES)
                    rows[e, sl] = rows[e, sl] * vb
            return 0

        lax.fori_loop(0, EDGE_BLK // LANES, gb, 0)

    for lc in range(cpc):
        if lc > 0:
            add_offset(rows_pad)
        c = cid * cpc + lc

        zero_rows0()
        rbase = sid * rpt
        nfull, rem = rpt // EDGE_BLK, rpt % EDGE_BLK
        for i in range(nfull):
            pltpu.sync_copy(rows0, acc.at[pl.ds(rbase + i * EDGE_BLK, EDGE_BLK)])
        if rem:
            pltpu.sync_copy(rows0.at[pl.ds(0, rem)],
                            acc.at[pl.ds(rbase + nfull * EDGE_BLK, rem)])
        plsc.subcore_barrier()

        def sidx(b):
            return src_a.at[pl.ds(b * EDGE_BLK, EDGE_BLK)]

        pltpu.async_copy(tableH.at[sidx(0)], rows0, gsem0)

        def pair(g, _):
            b0 = 2 * g
            b1 = b0 + 1
            pltpu.make_async_copy(tableH.at[sidx(b0)], rows0, gsem0).wait()
            pltpu.async_copy(tableH.at[sidx(b1)], rows1, gsem1)
            scale(rows0, b0)
            pltpu.sync_copy(rows0, acc.at[dst_a.at[b0]], add=True)
            pltpu.make_async_copy(tableH.at[sidx(b1)], rows1, gsem1).wait()

            @pl.when(g + 1 < nb_tile // 2)
            def _():
                pltpu.async_copy(tableH.at[sidx(b0 + 2)], rows0, gsem0)

            scale(rows1, b1)
            pltpu.sync_copy(rows1, acc.at[dst_a.at[b1]], add=True)
            return 0

        lax.fori_loop(0, nb_tile // 2, pair, 0)
        plsc.subcore_barrier()

        pltpu.sync_copy(acc.at[pl.ds(rbase, rpt)],
                        outH.at[pl.ds(c * rows_pad + rbase, rpt)])


@functools.lru_cache(maxsize=None)
def _get_spmm(rows_pad, nchunks, nb_tile):
    mesh = plsc.VectorSubcoreMesh(core_axis_name="c", subcore_axis_name="s")
    return pl.kernel(
        functools.partial(_spmm_body, rows_pad=rows_pad, nchunks=nchunks,
                          nb_tile=nb_tile),
        out_type=jax.ShapeDtypeStruct((nchunks * rows_pad, FCHUNK), jnp.float32),
        mesh=mesh,
        compiler_params=pltpu.CompilerParams(use_tc_tiling_on_sc=False),
        scratch_types=[
            pltpu.VMEM_SHARED((rows_pad, FCHUNK), jnp.float32),
            pltpu.VMEM((nb_tile * EDGE_BLK,), jnp.int32),
            pltpu.VMEM((nb_tile, EDGE_BLK), jnp.int32),
            pltpu.VMEM((nb_tile * EDGE_BLK,), jnp.float32),
            pltpu.VMEM((EDGE_BLK, FCHUNK), jnp.float32),
            pltpu.VMEM((EDGE_BLK, FCHUNK), jnp.float32),
            pltpu.SemaphoreType.DMA,
            pltpu.SemaphoreType.DMA,
        ],
    )


def _pad_edges(dst, src, val, e_pad):
    e = dst.shape[0]
    dst = jnp.pad(dst, (0, e_pad - e)).reshape(NTILES, -1, EDGE_BLK)
    src = jnp.pad(src, (0, e_pad - e))
    val = jnp.pad(val, (0, e_pad - e))
    return dst, src, val


def _spmm_sc(x, idx, val, rows, width):
    nchunks = width // FCHUNK
    grain_r = NTILES * 8
    rows_pad = ((rows + grain_r - 1) // grain_r) * grain_r
    grain = NTILES * EDGE_BLK
    e = val.shape[0]
    e_pad = ((e + 2 * grain - 1) // (2 * grain)) * (2 * grain)
    nb_tile = e_pad // NTILES // EDGE_BLK

    xp = jnp.pad(x, ((0, rows_pad - rows), (0, 0)))
    table = xp.reshape(rows_pad, nchunks, FCHUNK).transpose(1, 0, 2) \
              .reshape(nchunks * rows_pad, FCHUNK)
    dst2, src2, val2 = _pad_edges(idx[0], idx[1], val, e_pad)
    out = _get_spmm(rows_pad, nchunks, nb_tile)(table, src2, dst2, val2)
    return out.reshape(nchunks, rows_pad, FCHUNK)[:, :rows] \
              .transpose(1, 0, 2).reshape(rows, width)


def _bn(x, g, b):
    m = jnp.mean(x, axis=0)
    v = jnp.var(x, axis=0)
    return (x - m) / jnp.sqrt(v + 1e-5) * g + b


def _infonce(v1, v2, W, b, mask, n):
    v1 = v1 @ W + b
    v2 = v2 @ W + b
    v1 = v1 / jnp.linalg.norm(v1, axis=1, keepdims=True)
    v2 = v2 / jnp.linalg.norm(v2, axis=1, keepdims=True)
    pos = v1 @ v2.T / TEMP
    pos = jnp.where(mask[None, :], pos, -jnp.inf)
    score = jnp.diag(jax.nn.log_softmax(pos, axis=1))
    return -jnp.sum(jnp.where(mask, score, 0.0)) / n


def kernel(epoch, pro_idx, hpo_idx, X_exp, X_esm, X_ppi, X_term, A_ppi_idx, A_ppi_val, A_rel_idx, A_rel_val, A_cop_idx, A_cop_val, params):
    p = params
    pe = _bn(jax.nn.leaky_relu(X_exp @ p['W_exp'] + p['b_exp']), p['g_exp'], p['be_exp'])
    ps = _bn(jax.nn.leaky_relu(X_esm @ p['W_esm'] + p['b_esm']), p['g_esm'], p['be_esm'])
    pp = _bn(jax.nn.leaky_relu(X_ppi @ p['W_ppi'] + p['b_ppi']), p['g_ppi'], p['be_ppi'])
    t0 = _bn(jax.nn.leaky_relu(X_term @ p['W_pub0'] + p['b_pub0']), p['g_p0'], p['be_p0'])
    t1 = _bn(jax.nn.leaky_relu(X_term @ p['W_pub1'] + p['b_pub1']), p['g_p1'], p['be_p1'])
    t2 = _bn(jax.nn.leaky_relu(X_term @ p['W_pub2'] + p['b_pub2']), p['g_p2'], p['be_p2'])

    ego = jnp.concatenate([jnp.concatenate([pe, t0], axis=0),
                           jnp.concatenate([ps, t1], axis=0),
                           jnp.concatenate([pp, t2], axis=0)], axis=1)

    prop0 = _spmm_sc(ego, A_rel_idx, A_rel_val, N, 3 * D)
    prop1 = _spmm_sc(ego, A_cop_idx, A_cop_val, N, 3 * D)

    pe_f0, ps_f0, pp_f10 = prop0[:PRO, :D], prop0[:PRO, D:2 * D], prop0[:PRO, 2 * D:]
    te_f0, ts_f0, tp_f0 = prop0[PRO:, :D], prop0[PRO:, D:2 * D], prop0[PRO:, 2 * D:]
    pe_f1, ps_f1, pp_f11 = prop1[:PRO, :D], prop1[:PRO, D:2 * D], prop1[:PRO, 2 * D:]
    te_f1, ts_f1, tp_f1 = prop1[PRO:, :D], prop1[PRO:, D:2 * D], prop1[PRO:, 2 * D:]

    pp_stack = jnp.concatenate([pp_f10, pp_f11], axis=1)
    pp_f = _spmm_sc(pp_stack, A_ppi_idx, A_ppi_val, PRO, 2 * D)
    pp_f0, pp_f1 = pp_f[:, :D], pp_f[:, D:]

    pset, pcnt = jnp.unique(pro_idx, size=B, fill_value=0, return_counts=True)
    hset, hcnt = jnp.unique(hpo_idx, size=B, fill_value=0, return_counts=True)
    pmask = pcnt > 0
    hmask = hcnt > 0
    pn = jnp.sum(pmask)
    hn = jnp.sum(hmask)
    lp = (_infonce(pe_f0[pset], pe_f1[pset], p['W_pp'], p['b_pp'], pmask, pn)
          + _infonce(ps_f0[pset], ps_f1[pset], p['W_pp'], p['b_pp'], pmask, pn)
          + _infonce(pp_f0[pset], pp_f1[pset], p['W_pp'], p['b_pp'], pmask, pn)) / 3.0
    lt = (_infonce(te_f0[hset], te_f1[hset], p['W_pt'], p['b_pt'], hmask, hn)
          + _infonce(ts_f0[hset], ts_f1[hset], p['W_pt'], p['b_pt'], hmask, hn)
          + _infonce(tp_f0[hset], tp_f1[hset], p['W_pt'], p['b_pt'], hmask, hn)) / 3.0

    return (pe_f0, te_f0, ps_f0, ts_f0, pp_f0, tp_f0, pe, ps, pp, (lp + lt) / 2.0)


# speedup vs baseline: 2.0082x; 1.1000x over previous
import functools

import jax
import jax.numpy as jnp
from jax import lax
from jax.experimental import pallas as pl
from jax.experimental.pallas import tpu as pltpu
from jax.experimental.pallas import tpu_sc as plsc

PRO = 10000
TERM = 5000
N = PRO + TERM
D = 256
B = 4096
TEMP = 0.1

LANES = 16
NSC = 2
NTILES = 16
FCHUNK = 64
EDGE_BLK = 128

ROWS_PAD_N = 15104


SUPER = 384
SUBS = SUPER // EDGE_BLK


def _spmm_body(tableH, srcH, dstH, valH, outH, acc, dst_a, src_a0, src_a1,
               val_a0, val_a1, r00, r01, r02, r10, r11, r12,
               gsem0, gsem1, ssem0, ssem1, isem0, isem1,
               *, rows_pad, nchunks, nb_tile):
    cpc = nchunks // NSC
    rpt = rows_pad // NTILES
    npt = nb_tile * EDGE_BLK
    ns = npt // SUPER
    cid = lax.axis_index("c")
    sid = lax.axis_index("s")
    src_a = (src_a0, src_a1)
    val_a = (val_a0, val_a1)
    rows = ((r00, r01, r02), (r10, r11, r12))
    gsem = (gsem0, gsem1)
    ssem = (ssem0, ssem1)
    isem = (isem0, isem1)

    pltpu.sync_copy(dstH.at[sid], dst_a)

    def sv_copy(s, u, sync=False):
        off = sid * npt + s * SUPER
        if sync:
            pltpu.sync_copy(srcH.at[pl.ds(off, SUPER)], src_a[u])
            pltpu.sync_copy(valH.at[pl.ds(off, SUPER)], val_a[u])
        else:
            pltpu.async_copy(srcH.at[pl.ds(off, SUPER)], src_a[u], isem[u])
            pltpu.async_copy(valH.at[pl.ds(off, SUPER)], val_a[u], isem[u])

    def sv_wait(s, u):
        off = sid * npt + s * SUPER
        pltpu.make_async_copy(srcH.at[pl.ds(off, SUPER)], src_a[u], isem[u]).wait()
        pltpu.make_async_copy(valH.at[pl.ds(off, SUPER)], val_a[u], isem[u]).wait()

    def add_offset(u, delta):
        dvec = jnp.full((LANES,), delta, jnp.int32)

        def ob(i, _):
            sl = pl.ds(i * LANES, LANES)
            src_a[u][sl] = src_a[u][sl] + dvec
            return 0

        lax.fori_loop(0, SUPER // LANES, ob, 0)

    def fire_gather(u):
        for j in range(SUBS):
            pltpu.async_copy(tableH.at[src_a[u].at[pl.ds(j * EDGE_BLK, EDGE_BLK)]],
                             rows[u][j], gsem[u])

    def wait_gather(u):
        for j in range(SUBS):
            pltpu.make_async_copy(
                tableH.at[src_a[u].at[pl.ds(j * EDGE_BLK, EDGE_BLK)]],
                rows[u][j], gsem[u]).wait()

    def fire_scatter(s, u):
        for j in range(SUBS):
            pltpu.async_copy(rows[u][j], acc.at[dst_a.at[s * SUBS + j]],
                             ssem[u], add=True)

    def wait_scatter(s, u):
        for j in range(SUBS):
            pltpu.make_async_copy(rows[u][j], acc.at[dst_a.at[s * SUBS + j]],
                                  ssem[u]).wait()

    def zero_rows0():
        def zb(r, _):
            for q in range(FCHUNK // LANES):
                r00[r, pl.ds(q * LANES, LANES)] = jnp.zeros((LANES,), jnp.float32)
            return 0

        lax.fori_loop(0, EDGE_BLK, zb, 0)

    _gdn = lax.GatherDimensionNumbers(offset_dims=(), collapsed_slice_dims=(0,),
                                      start_index_map=(0,))

    def scale(u):
        for j in range(SUBS):
            rbuf = rows[u][j]

            def gb(g, _):
                v16 = val_a[u][pl.ds(j * EDGE_BLK + g * LANES, LANES)]
                for k in range(LANES):
                    vb = lax.gather(v16, jnp.full((LANES, 1), k, jnp.int32), _gdn,
                                    slice_sizes=(1,),
                                    mode=lax.GatherScatterMode.PROMISE_IN_BOUNDS)
                    e = g * LANES + k
                    for q in range(FCHUNK // LANES):
                        sl = pl.ds(q * LANES, LANES)
                        rbuf[e, sl] = rbuf[e, sl] * vb
                return 0

            lax.fori_loop(0, EDGE_BLK // LANES, gb, 0)

    for lc in range(cpc):
        c = cid * cpc + lc
        coff = c * rows_pad

        zero_rows0()
        rbase = sid * rpt
        nfull, rem = rpt // EDGE_BLK, rpt % EDGE_BLK
        for i in range(nfull):
            pltpu.sync_copy(r00, acc.at[pl.ds(rbase + i * EDGE_BLK, EDGE_BLK)])
        if rem:
            pltpu.sync_copy(r00.at[pl.ds(0, rem)],
                            acc.at[pl.ds(rbase + nfull * EDGE_BLK, rem)])
        plsc.subcore_barrier()

        sv_copy(0, 0, sync=True)
        add_offset(0, coff)
        fire_gather(0)
        if ns > 1:
            sv_copy(1, 1)

        def step(s, _):
            u = lax.rem(s, 2)

            def even(su):
                uu, oo = su
                wait_gather(uu)

                @pl.when(s > 0)
                def _():
                    wait_scatter(s - 1, oo)

                @pl.when(s + 1 < ns)
                def _():
                    sv_wait(s + 1, oo)
                    add_offset(oo, coff)
                    fire_gather(oo)

                scale(uu)
                fire_scatter(s, uu)

                @pl.when(s + 2 < ns)
                def _():
                    sv_copy(s + 2, uu)

            lax.cond(u == 0, lambda: even((0, 1)), lambda: even((1, 0)))
            return 0

        lax.fori_loop(0, ns, step, 0)
        wait_scatter(ns - 1, (ns - 1) % 2)
        plsc.subcore_barrier()

        pltpu.sync_copy(acc.at[pl.ds(rbase, rpt)],
                        outH.at[pl.ds(coff + rbase, rpt)])


@functools.lru_cache(maxsize=None)
def _get_spmm(rows_pad, nchunks, nb_tile):
    mesh = plsc.VectorSubcoreMesh(core_axis_name="c", subcore_axis_name="s")
    return pl.kernel(
        functools.partial(_spmm_body, rows_pad=rows_pad, nchunks=nchunks,
                          nb_tile=nb_tile),
        out_type=jax.ShapeDtypeStruct((nchunks * rows_pad, FCHUNK), jnp.float32),
        mesh=mesh,
        compiler_params=pltpu.CompilerParams(use_tc_tiling_on_sc=False),
        scratch_types=[
            pltpu.VMEM_SHARED((rows_pad, FCHUNK), jnp.float32),
            pltpu.VMEM((nb_tile, EDGE_BLK), jnp.int32),
            pltpu.VMEM((SUPER,), jnp.int32),
            pltpu.VMEM((SUPER,), jnp.int32),
            pltpu.VMEM((SUPER,), jnp.float32),
            pltpu.VMEM((SUPER,), jnp.float32),
            pltpu.VMEM((EDGE_BLK, FCHUNK), jnp.float32),
            pltpu.VMEM((EDGE_BLK, FCHUNK), jnp.float32),
            pltpu.VMEM((EDGE_BLK, FCHUNK), jnp.float32),
            pltpu.VMEM((EDGE_BLK, FCHUNK), jnp.float32),
            pltpu.VMEM((EDGE_BLK, FCHUNK), jnp.float32),
            pltpu.VMEM((EDGE_BLK, FCHUNK), jnp.float32),
            pltpu.SemaphoreType.DMA,
            pltpu.SemaphoreType.DMA,
            pltpu.SemaphoreType.DMA,
            pltpu.SemaphoreType.DMA,
            pltpu.SemaphoreType.DMA,
            pltpu.SemaphoreType.DMA,
        ],
    )


def _pad_edges(dst, src, val, e_pad):
    e = dst.shape[0]
    dst = jnp.pad(dst, (0, e_pad - e)).reshape(NTILES, -1, EDGE_BLK)
    src = jnp.pad(src, (0, e_pad - e))
    val = jnp.pad(val, (0, e_pad - e))
    return dst, src, val


def _spmm_sc(x, idx, val, rows, width):
    nchunks = width // FCHUNK
    grain_r = NTILES * 8
    rows_pad = ((rows + grain_r - 1) // grain_r) * grain_r
    grain = NTILES * SUPER
    e = val.shape[0]
    e_pad = ((e + grain - 1) // grain) * grain
    nb_tile = e_pad // NTILES // EDGE_BLK

    xp = jnp.pad(x, ((0, rows_pad - rows), (0, 0)))
    table = xp.reshape(rows_pad, nchunks, FCHUNK).transpose(1, 0, 2) \
              .reshape(nchunks * rows_pad, FCHUNK)
    dst2, src2, val2 = _pad_edges(idx[0], idx[1], val, e_pad)
    out = _get_spmm(rows_pad, nchunks, nb_tile)(table, src2, dst2, val2)
    return out.reshape(nchunks, rows_pad, FCHUNK)[:, :rows] \
              .transpose(1, 0, 2).reshape(rows, width)


def _bn(x, g, b):
    m = jnp.mean(x, axis=0)
    v = jnp.var(x, axis=0)
    return (x - m) / jnp.sqrt(v + 1e-5) * g + b


def _infonce(v1, v2, W, b, mask, n):
    v1 = v1 @ W + b
    v2 = v2 @ W + b
    v1 = v1 / jnp.linalg.norm(v1, axis=1, keepdims=True)
    v2 = v2 / jnp.linalg.norm(v2, axis=1, keepdims=True)
    pos = v1 @ v2.T / TEMP
    pos = jnp.where(mask[None, :], pos, -jnp.inf)
    score = jnp.diag(jax.nn.log_softmax(pos, axis=1))
    return -jnp.sum(jnp.where(mask, score, 0.0)) / n


def kernel(epoch, pro_idx, hpo_idx, X_exp, X_esm, X_ppi, X_term, A_ppi_idx, A_ppi_val, A_rel_idx, A_rel_val, A_cop_idx, A_cop_val, params):
    p = params
    pe = _bn(jax.nn.leaky_relu(X_exp @ p['W_exp'] + p['b_exp']), p['g_exp'], p['be_exp'])
    ps = _bn(jax.nn.leaky_relu(X_esm @ p['W_esm'] + p['b_esm']), p['g_esm'], p['be_esm'])
    pp = _bn(jax.nn.leaky_relu(X_ppi @ p['W_ppi'] + p['b_ppi']), p['g_ppi'], p['be_ppi'])
    t0 = _bn(jax.nn.leaky_relu(X_term @ p['W_pub0'] + p['b_pub0']), p['g_p0'], p['be_p0'])
    t1 = _bn(jax.nn.leaky_relu(X_term @ p['W_pub1'] + p['b_pub1']), p['g_p1'], p['be_p1'])
    t2 = _bn(jax.nn.leaky_relu(X_term @ p['W_pub2'] + p['b_pub2']), p['g_p2'], p['be_p2'])

    ego = jnp.concatenate([jnp.concatenate([pe, t0], axis=0),
                           jnp.concatenate([ps, t1], axis=0),
                           jnp.concatenate([pp, t2], axis=0)], axis=1)

    prop0 = _spmm_sc(ego, A_rel_idx, A_rel_val, N, 3 * D)
    prop1 = _spmm_sc(ego, A_cop_idx, A_cop_val, N, 3 * D)

    pe_f0, ps_f0, pp_f10 = prop0[:PRO, :D], prop0[:PRO, D:2 * D], prop0[:PRO, 2 * D:]
    te_f0, ts_f0, tp_f0 = prop0[PRO:, :D], prop0[PRO:, D:2 * D], prop0[PRO:, 2 * D:]
    pe_f1, ps_f1, pp_f11 = prop1[:PRO, :D], prop1[:PRO, D:2 * D], prop1[:PRO, 2 * D:]
    te_f1, ts_f1, tp_f1 = prop1[PRO:, :D], prop1[PRO:, D:2 * D], prop1[PRO:, 2 * D:]

    pp_stack = jnp.concatenate([pp_f10, pp_f11], axis=1)
    pp_f = _spmm_sc(pp_stack, A_ppi_idx, A_ppi_val, PRO, 2 * D)
    pp_f0, pp_f1 = pp_f[:, :D], pp_f[:, D:]

    pset, pcnt = jnp.unique(pro_idx, size=B, fill_value=0, return_counts=True)
    hset, hcnt = jnp.unique(hpo_idx, size=B, fill_value=0, return_counts=True)
    pmask = pcnt > 0
    hmask = hcnt > 0
    pn = jnp.sum(pmask)
    hn = jnp.sum(hmask)
    lp = (_infonce(pe_f0[pset], pe_f1[pset], p['W_pp'], p['b_pp'], pmask, pn)
          + _infonce(ps_f0[pset], ps_f1[pset], p['W_pp'], p['b_pp'], pmask, pn)
          + _infonce(pp_f0[pset], pp_f1[pset], p['W_pp'], p['b_pp'], pmask, pn)) / 3.0
    lt = (_infonce(te_f0[hset], te_f1[hset], p['W_pt'], p['b_pt'], hmask, hn)
          + _infonce(ts_f0[hset], ts_f1[hset], p['W_pt'], p['b_pt'], hmask, hn)
          + _infonce(tp_f0[hset], tp_f1[hset], p['W_pt'], p['b_pt'], hmask, hn)) / 3.0

    return (pe_f0, te_f0, ps_f0, ts_f0, pp_f0, tp_f0, pe, ps, pp, (lp + lt) / 2.0)


# speedup vs baseline: 2.4203x; 1.2052x over previous
import functools

import jax
import jax.numpy as jnp
from jax import lax
from jax.experimental import pallas as pl
from jax.experimental.pallas import tpu as pltpu
from jax.experimental.pallas import tpu_sc as plsc

PRO = 10000
TERM = 5000
N = PRO + TERM
D = 256
B = 4096
TEMP = 0.1

LANES = 16
NSC = 2
NTILES = 16
FCHUNK = 64
EDGE_BLK = 128

ROWS_PAD_N = 15104


SUPER = 256
SUBS = SUPER // EDGE_BLK


def _spmm_body(tableH, srcH, dstH, valH, outH, acc, dst_a, src_a0, src_a1,
               val_a0, val_a1, r00, r01, r10, r11,
               gsem0, gsem1, ssem0, ssem1, isem0, isem1,
               *, rows_pad, nchunks, nb_tile):
    cpc = nchunks // NSC
    rpt = rows_pad // NTILES
    npt = nb_tile * EDGE_BLK
    ns = npt // SUPER
    cid = lax.axis_index("c")
    sid = lax.axis_index("s")
    src_a = (src_a0, src_a1)
    val_a = (val_a0, val_a1)
    rows = ((r00, r01), (r10, r11))
    gsem = (gsem0, gsem1)
    ssem = (ssem0, ssem1)
    isem = (isem0, isem1)

    pltpu.sync_copy(dstH.at[sid], dst_a)

    def sv_copy(s, u, sync=False):
        off = sid * npt + s * SUPER
        if sync:
            pltpu.sync_copy(srcH.at[pl.ds(off, SUPER)], src_a[u])
            pltpu.sync_copy(valH.at[pl.ds(off * LANES, SUPER * LANES)], val_a[u])
        else:
            pltpu.async_copy(srcH.at[pl.ds(off, SUPER)], src_a[u], isem[u])
            pltpu.async_copy(valH.at[pl.ds(off * LANES, SUPER * LANES)],
                             val_a[u], isem[u])

    def sv_wait(s, u):
        off = sid * npt + s * SUPER
        pltpu.make_async_copy(srcH.at[pl.ds(off, SUPER)], src_a[u], isem[u]).wait()
        pltpu.make_async_copy(valH.at[pl.ds(off * LANES, SUPER * LANES)],
                              val_a[u], isem[u]).wait()

    def add_offset(u, delta):
        dvec = jnp.full((LANES,), delta, jnp.int32)

        def ob(i, _):
            sl = pl.ds(i * LANES, LANES)
            src_a[u][sl] = src_a[u][sl] + dvec
            return 0

        lax.fori_loop(0, SUPER // LANES, ob, 0)

    def fire_gather(u):
        for j in range(SUBS):
            pltpu.async_copy(tableH.at[src_a[u].at[pl.ds(j * EDGE_BLK, EDGE_BLK)]],
                             rows[u][j], gsem[u])

    def wait_gather(u):
        for j in range(SUBS):
            pltpu.make_async_copy(
                tableH.at[src_a[u].at[pl.ds(j * EDGE_BLK, EDGE_BLK)]],
                rows[u][j], gsem[u]).wait()

    def fire_scatter(s, u):
        for j in range(SUBS):
            pltpu.async_copy(rows[u][j], acc.at[dst_a.at[s * SUBS + j]],
                             ssem[u], add=True)

    def wait_scatter(s, u):
        for j in range(SUBS):
            pltpu.make_async_copy(rows[u][j], acc.at[dst_a.at[s * SUBS + j]],
                                  ssem[u]).wait()

    def zero_rows0():
        def zb(r, _):
            for q in range(FCHUNK // LANES):
                r00[r, pl.ds(q * LANES, LANES)] = jnp.zeros((LANES,), jnp.float32)
            return 0

        lax.fori_loop(0, EDGE_BLK, zb, 0)

    def scale(u):
        for j in range(SUBS):
            rbuf = rows[u][j]

            def eb(e, _):
                vb = val_a[u][pl.ds((j * EDGE_BLK + e) * LANES, LANES)]
                for q in range(FCHUNK // LANES):
                    sl = pl.ds(q * LANES, LANES)
                    rbuf[e, sl] = rbuf[e, sl] * vb
                return 0

            lax.fori_loop(0, EDGE_BLK, eb, 0, unroll=8)

    for lc in range(cpc):
        c = cid * cpc + lc
        coff = c * rows_pad

        zero_rows0()
        rbase = sid * rpt
        nfull, rem = rpt // EDGE_BLK, rpt % EDGE_BLK
        for i in range(nfull):
            pltpu.sync_copy(r00, acc.at[pl.ds(rbase + i * EDGE_BLK, EDGE_BLK)])
        if rem:
            pltpu.sync_copy(r00.at[pl.ds(0, rem)],
                            acc.at[pl.ds(rbase + nfull * EDGE_BLK, rem)])
        plsc.subcore_barrier()

        sv_copy(0, 0, sync=True)
        add_offset(0, coff)
        fire_gather(0)
        if ns > 1:
            sv_copy(1, 1)

        def step(s, _):
            u = lax.rem(s, 2)

            def even(su):
                uu, oo = su
                wait_gather(uu)

                @pl.when(s > 0)
                def _():
                    wait_scatter(s - 1, oo)

                @pl.when(s + 1 < ns)
                def _():
                    sv_wait(s + 1, oo)
                    add_offset(oo, coff)
                    fire_gather(oo)

                scale(uu)
                fire_scatter(s, uu)

                @pl.when(s + 2 < ns)
                def _():
                    sv_copy(s + 2, uu)

            lax.cond(u == 0, lambda: even((0, 1)), lambda: even((1, 0)))
            return 0

        lax.fori_loop(0, ns, step, 0)
        wait_scatter(ns - 1, (ns - 1) % 2)
        plsc.subcore_barrier()

        pltpu.sync_copy(acc.at[pl.ds(rbase, rpt)],
                        outH.at[pl.ds(coff + rbase, rpt)])


@functools.lru_cache(maxsize=None)
def _get_spmm(rows_pad, nchunks, nb_tile):
    mesh = plsc.VectorSubcoreMesh(core_axis_name="c", subcore_axis_name="s")
    return pl.kernel(
        functools.partial(_spmm_body, rows_pad=rows_pad, nchunks=nchunks,
                          nb_tile=nb_tile),
        out_type=jax.ShapeDtypeStruct((nchunks * rows_pad, FCHUNK), jnp.float32),
        mesh=mesh,
        compiler_params=pltpu.CompilerParams(use_tc_tiling_on_sc=False),
        scratch_types=[
            pltpu.VMEM_SHARED((rows_pad, FCHUNK), jnp.float32),
            pltpu.VMEM((nb_tile, EDGE_BLK), jnp.int32),
            pltpu.VMEM((SUPER,), jnp.int32),
            pltpu.VMEM((SUPER,), jnp.int32),
            pltpu.VMEM((SUPER * LANES,), jnp.float32),
            pltpu.VMEM((SUPER * LANES,), jnp.float32),
            pltpu.VMEM((EDGE_BLK, FCHUNK), jnp.float32),
            pltpu.VMEM((EDGE_BLK, FCHUNK), jnp.float32),
            pltpu.VMEM((EDGE_BLK, FCHUNK), jnp.float32),
            pltpu.VMEM((EDGE_BLK, FCHUNK), jnp.float32),
            pltpu.SemaphoreType.DMA,
            pltpu.SemaphoreType.DMA,
            pltpu.SemaphoreType.DMA,
            pltpu.SemaphoreType.DMA,
            pltpu.SemaphoreType.DMA,
            pltpu.SemaphoreType.DMA,
        ],
    )


def _pad_edges(dst, src, val, e_pad):
    e = dst.shape[0]
    dst = jnp.pad(dst, (0, e_pad - e)).reshape(NTILES, -1, EDGE_BLK)
    src = jnp.pad(src, (0, e_pad - e))
    val = jnp.pad(val, (0, e_pad - e))
    val = jnp.broadcast_to(val[:, None], (e_pad, LANES)).reshape(-1)
    return dst, src, val


def _spmm_sc(x, idx, val, rows, width):
    nchunks = width // FCHUNK
    grain_r = NTILES * 8
    rows_pad = ((rows + grain_r - 1) // grain_r) * grain_r
    grain = NTILES * SUPER
    e = val.shape[0]
    e_pad = ((e + grain - 1) // grain) * grain
    nb_tile = e_pad // NTILES // EDGE_BLK

    xp = jnp.pad(x, ((0, rows_pad - rows), (0, 0)))
    table = xp.reshape(rows_pad, nchunks, FCHUNK).transpose(1, 0, 2) \
              .reshape(nchunks * rows_pad, FCHUNK)
    dst2, src2, val2 = _pad_edges(idx[0], idx[1], val, e_pad)
    out = _get_spmm(rows_pad, nchunks, nb_tile)(table, src2, dst2, val2)
    return out.reshape(nchunks, rows_pad, FCHUNK)[:, :rows] \
              .transpose(1, 0, 2).reshape(rows, width)


def _bn(x, g, b):
    m = jnp.mean(x, axis=0)
    v = jnp.var(x, axis=0)
    return (x - m) / jnp.sqrt(v + 1e-5) * g + b


def _infonce(v1, v2, W, b, mask, n):
    v1 = v1 @ W + b
    v2 = v2 @ W + b
    v1 = v1 / jnp.linalg.norm(v1, axis=1, keepdims=True)
    v2 = v2 / jnp.linalg.norm(v2, axis=1, keepdims=True)
    pos = v1 @ v2.T / TEMP
    pos = jnp.where(mask[None, :], pos, -jnp.inf)
    score = jnp.diag(jax.nn.log_softmax(pos, axis=1))
    return -jnp.sum(jnp.where(mask, score, 0.0)) / n


def kernel(epoch, pro_idx, hpo_idx, X_exp, X_esm, X_ppi, X_term, A_ppi_idx, A_ppi_val, A_rel_idx, A_rel_val, A_cop_idx, A_cop_val, params):
    p = params
    pe = _bn(jax.nn.leaky_relu(X_exp @ p['W_exp'] + p['b_exp']), p['g_exp'], p['be_exp'])
    ps = _bn(jax.nn.leaky_relu(X_esm @ p['W_esm'] + p['b_esm']), p['g_esm'], p['be_esm'])
    pp = _bn(jax.nn.leaky_relu(X_ppi @ p['W_ppi'] + p['b_ppi']), p['g_ppi'], p['be_ppi'])
    t0 = _bn(jax.nn.leaky_relu(X_term @ p['W_pub0'] + p['b_pub0']), p['g_p0'], p['be_p0'])
    t1 = _bn(jax.nn.leaky_relu(X_term @ p['W_pub1'] + p['b_pub1']), p['g_p1'], p['be_p1'])
    t2 = _bn(jax.nn.leaky_relu(X_term @ p['W_pub2'] + p['b_pub2']), p['g_p2'], p['be_p2'])

    ego = jnp.concatenate([jnp.concatenate([pe, t0], axis=0),
                           jnp.concatenate([ps, t1], axis=0),
                           jnp.concatenate([pp, t2], axis=0)], axis=1)

    prop0 = _spmm_sc(ego, A_rel_idx, A_rel_val, N, 3 * D)
    prop1 = _spmm_sc(ego, A_cop_idx, A_cop_val, N, 3 * D)

    pe_f0, ps_f0, pp_f10 = prop0[:PRO, :D], prop0[:PRO, D:2 * D], prop0[:PRO, 2 * D:]
    te_f0, ts_f0, tp_f0 = prop0[PRO:, :D], prop0[PRO:, D:2 * D], prop0[PRO:, 2 * D:]
    pe_f1, ps_f1, pp_f11 = prop1[:PRO, :D], prop1[:PRO, D:2 * D], prop1[:PRO, 2 * D:]
    te_f1, ts_f1, tp_f1 = prop1[PRO:, :D], prop1[PRO:, D:2 * D], prop1[PRO:, 2 * D:]

    pp_stack = jnp.concatenate([pp_f10, pp_f11], axis=1)
    pp_f = _spmm_sc(pp_stack, A_ppi_idx, A_ppi_val, PRO, 2 * D)
    pp_f0, pp_f1 = pp_f[:, :D], pp_f[:, D:]

    pset, pcnt = jnp.unique(pro_idx, size=B, fill_value=0, return_counts=True)
    hset, hcnt = jnp.unique(hpo_idx, size=B, fill_value=0, return_counts=True)
    pmask = pcnt > 0
    hmask = hcnt > 0
    pn = jnp.sum(pmask)
    hn = jnp.sum(hmask)
    lp = (_infonce(pe_f0[pset], pe_f1[pset], p['W_pp'], p['b_pp'], pmask, pn)
          + _infonce(ps_f0[pset], ps_f1[pset], p['W_pp'], p['b_pp'], pmask, pn)
          + _infonce(pp_f0[pset], pp_f1[pset], p['W_pp'], p['b_pp'], pmask, pn)) / 3.0
    lt = (_infonce(te_f0[hset], te_f1[hset], p['W_pt'], p['b_pt'], hmask, hn)
          + _infonce(ts_f0[hset], ts_f1[hset], p['W_pt'], p['b_pt'], hmask, hn)
          + _infonce(tp_f0[hset], tp_f1[hset], p['W_pt'], p['b_pt'], hmask, hn)) / 3.0

    return (pe_f0, te_f0, ps_f0, ts_f0, pp_f0, tp_f0, pe, ps, pp, (lp + lt) / 2.0)
